# Initial kernel scaffold; baseline (speedup 1.0000x reference)
import jax
import jax.numpy as jnp
from jax.experimental import pallas as pl


def kernel(x, edge_indices, edge_values, embedding, Wq, Wk, Wv, W, b):
    raise NotImplementedError("write your pallas kernel here")


# speedup vs baseline: 8.0589x; 8.0589x over previous
import functools

import jax
import jax.numpy as jnp
from jax import lax
from jax.experimental import pallas as pl
from jax.experimental.pallas import tpu as pltpu
from jax.experimental.pallas import tpu_sc as plsc

N_NODES = 10000
E = 320000
D = 128
B = 16384
LAMDA = 0.8

NC = 2
NS = 16
NW = NC * NS
CH = 128

NCH1 = 79
EPW = NCH1 * CH
EPAD = NW * EPW
EPT3 = EPAD // NS
NCH3 = EPT3 // CH
N16 = 10240

_INV_SQRT_D = 1.0 / (D ** 0.5)

_mesh = plsc.VectorSubcoreMesh(core_axis_name="c", subcore_axis_name="s")


@functools.partial(
    pl.kernel,
    out_type=[
        jax.ShapeDtypeStruct((EPAD,), jnp.float32),
        jax.ShapeDtypeStruct((NC * N16,), jnp.float32),
    ],
    mesh=_mesh,
    compiler_params=pltpu.CompilerParams(needs_layout_passes=False),
    scratch_types=[
        pltpu.VMEM((CH,), jnp.int32),
        pltpu.VMEM((CH,), jnp.int32),
        pltpu.VMEM((CH, D), jnp.float32),
        pltpu.VMEM((CH, D), jnp.float32),
        pltpu.VMEM((CH,), jnp.float32),
        pltpu.VMEM((CH,), jnp.float32),
        pltpu.VMEM_SHARED((N16,), jnp.float32),
        pltpu.SemaphoreType.DMA,
        pltpu.SemaphoreType.DMA,
    ],
)
def _p1(q_hbm, k_hbm, src_hbm, dst_hbm, ev_hbm, zn_hbm,
        ew_hbm, deg_hbm,
        dsti, srci, qr, kr, ewv, evv, deg_sh, sem1, sem2):
    c = lax.axis_index("c")
    s = lax.axis_index("s")
    wid = s * NC + c

    pltpu.sync_copy(zn_hbm.at[pl.ds(s * 640, 640)],
                    deg_sh.at[pl.ds(s * 640, 640)])
    plsc.subcore_barrier()

    iota = lax.iota(jnp.int32, 16)

    def chunk_body(ci, _):
        base = wid * EPW + ci * CH
        pltpu.sync_copy(dst_hbm.at[pl.ds(base, CH)], dsti)
        pltpu.sync_copy(src_hbm.at[pl.ds(base, CH)], srci)
        pltpu.sync_copy(ev_hbm.at[pl.ds(base, CH)], evv)
        cp_q = pltpu.async_copy(q_hbm.at[dsti], qr, sem1)
        cp_k = pltpu.async_copy(k_hbm.at[srci], kr, sem2)
        cp_q.wait()
        cp_k.wait()

        def grp_body(g, _):
            lg = jnp.zeros((16,), jnp.float32)
            for i in range(16):
                e = g * 16 + i
                acc = qr[e, pl.ds(0, 16)] * kr[e, pl.ds(0, 16)]
                for j in range(1, D // 16):
                    sl = pl.ds(j * 16, 16)
                    acc = acc + qr[e, sl] * kr[e, sl]
                s = jnp.sum(acc)
                lg = jnp.where(iota == i, s, lg)
            ew = jnp.exp(lg * _INV_SQRT_D)
            eid = base + g * 16 + iota
            ew = jnp.where(eid < E, ew, 0.0)
            ewv[pl.ds(g * 16, 16)] = ew
            return 0

        lax.fori_loop(0, CH // 16, grp_body, 0)
        pltpu.sync_copy(evv, deg_sh.at[dsti], add=True)
        pltpu.sync_copy(ewv, ew_hbm.at[pl.ds(base, CH)])
        return 0

    lax.fori_loop(0, NCH1, chunk_body, 0)

    plsc.subcore_barrier()
    pltpu.sync_copy(deg_sh.at[pl.ds(s * 640, 640)],
                    deg_hbm.at[pl.ds(c * N16 + s * 640, 640)])


@functools.partial(
    pl.kernel,
    out_type=[
        jax.ShapeDtypeStruct((N16, D), jnp.float32),
        jax.ShapeDtypeStruct((N16,), jnp.float32),
        jax.ShapeDtypeStruct((N16, D), jnp.float32),
    ],
    mesh=_mesh,
    compiler_params=pltpu.CompilerParams(needs_layout_passes=False),
    scratch_types=[
        pltpu.VMEM((CH,), jnp.int32),
        pltpu.VMEM((CH,), jnp.int32),
        pltpu.VMEM((CH,), jnp.float32),
        pltpu.VMEM((CH,), jnp.float32),
        pltpu.VMEM((CH, D), jnp.float32),
        pltpu.VMEM_SHARED((N16, D), jnp.float32),
        pltpu.VMEM_SHARED((N16,), jnp.float32),
        pltpu.SemaphoreType.DMA,
        pltpu.SemaphoreType.DMA,
    ],
)
def _p3(v_hbm, h_hbm, ew_hbm, src_hbm, dst_hbm, ev_hbm, dis_hbm,
        znd_hbm, zn_hbm,
        numer_hbm, den_hbm, agg_hbm,
        dsti, srci, wv, auxv, rows, rows_sh, den_sh, sem1, sem2):
    c = lax.axis_index("c")
    s = lax.axis_index("s")

    pltpu.sync_copy(znd_hbm.at[pl.ds(s * 640, 640)],
                    rows_sh.at[pl.ds(s * 640, 640)])
    pltpu.sync_copy(zn_hbm.at[pl.ds(s * 640, 640)],
                    den_sh.at[pl.ds(s * 640, 640)])
    plsc.subcore_barrier()

    def scale_rows(_=None):
        def g_body(g, _):
            w16 = wv[pl.ds(g * 16, 16)]
            for i in range(16):
                e = g * 16 + i
                w = w16[i]
                for j in range(D // 16):
                    sl = pl.ds(j * 16, 16)
                    rows[e, sl] = rows[e, sl] * w
            return 0
        lax.fori_loop(0, CH // 16, g_body, 0)

    def attn_chunk(ci, _):
        base = s * EPT3 + ci * CH
        pltpu.sync_copy(dst_hbm.at[pl.ds(base, CH)], dsti)
        pltpu.sync_copy(src_hbm.at[pl.ds(base, CH)], srci)
        pltpu.sync_copy(ew_hbm.at[pl.ds(base, CH)], wv)
        pltpu.async_copy(v_hbm.at[srci], rows, sem1).wait()
        scale_rows()
        pltpu.sync_copy(rows, rows_sh.at[dsti], add=True)
        pltpu.sync_copy(wv, den_sh.at[dsti], add=True)
        return 0

    def gcn_chunk(ci, _):
        base = s * EPT3 + ci * CH
        pltpu.sync_copy(dst_hbm.at[pl.ds(base, CH)], dsti)
        pltpu.sync_copy(src_hbm.at[pl.ds(base, CH)], srci)
        pltpu.sync_copy(ev_hbm.at[pl.ds(base, CH)], auxv)
        cp_r = pltpu.async_copy(h_hbm.at[srci], rows, sem1)
        cp_d = pltpu.async_copy(dis_hbm.at[srci], wv, sem2)
        cp_r.wait()
        cp_d.wait()
        for g in range(CH // 16):
            sl = pl.ds(g * 16, 16)
            wv[sl] = wv[sl] * auxv[sl]
        scale_rows()
        pltpu.sync_copy(rows, rows_sh.at[dsti], add=True)
        return 0

    @pl.when(c == 0)
    def _():
        lax.fori_loop(0, NCH3, attn_chunk, 0)

    @pl.when(c == 1)
    def _():
        lax.fori_loop(0, NCH3, gcn_chunk, 0)

    plsc.subcore_barrier()

    @pl.when(c == 0)
    def _():
        pltpu.sync_copy(rows_sh.at[pl.ds(s * 640, 640)],
                        numer_hbm.at[pl.ds(s * 640, 640)])
        pltpu.sync_copy(den_sh.at[pl.ds(s * 640, 640)],
                        den_hbm.at[pl.ds(s * 640, 640)])

    @pl.when(c == 1)
    def _():
        pltpu.sync_copy(rows_sh.at[pl.ds(s * 640, 640)],
                        agg_hbm.at[pl.ds(s * 640, 640)])


_BPW = B // NW
_BCH = _BPW // CH


@functools.partial(
    pl.kernel,
    out_type=jax.ShapeDtypeStruct((B, D), jnp.float32),
    mesh=_mesh,
    compiler_params=pltpu.CompilerParams(needs_layout_passes=False),
    scratch_types=[
        pltpu.VMEM((CH,), jnp.int32),
        pltpu.VMEM((CH, D), jnp.float32),
        pltpu.SemaphoreType.DMA,
    ],
)
def _p5(emb_hbm, x_hbm, out_hbm, xi, rows, sem):
    c = lax.axis_index("c")
    s = lax.axis_index("s")
    wid = s * NC + c

    def chunk_body(ci, _):
        base = wid * _BPW + ci * CH
        pltpu.sync_copy(x_hbm.at[pl.ds(base, CH)], xi)
        pltpu.async_copy(emb_hbm.at[xi], rows, sem).wait()
        pltpu.sync_copy(rows, out_hbm.at[pl.ds(base, CH)])
        return 0

    lax.fori_loop(0, _BCH, chunk_body, 0)


def _mm_body(a_ref, w_ref, o_ref):
    o_ref[...] = jnp.dot(a_ref[...], w_ref[...],
                         preferred_element_type=jnp.float32)


def _matmul4(embedding, w4):
    grid = (N_NODES // 400,)
    return pl.pallas_call(
        _mm_body,
        grid=grid,
        in_specs=[
            pl.BlockSpec((400, D), lambda i: (i, 0)),
            pl.BlockSpec((D, 4 * D), lambda i: (0, 0)),
        ],
        out_specs=pl.BlockSpec((400, 4 * D), lambda i: (i, 0)),
        out_shape=jax.ShapeDtypeStruct((N_NODES, 4 * D), jnp.float32),
    )(embedding, w4)


def _dis_body(deg_ref, dis_ref):
    d = deg_ref[0, :] + deg_ref[1, :]
    dis_ref[0, :] = jnp.where(d > 0, lax.rsqrt(jnp.where(d > 0, d, 1.0)), 0.0)


def _compute_dis(deg2):
    return pl.pallas_call(
        _dis_body,
        out_shape=jax.ShapeDtypeStruct((1, N16), jnp.float32),
    )(deg2)


def _emb_body(num_ref, den_ref, agg_ref, h_ref, dis_ref, b_ref, o_ref):
    o_ref[...] = (num_ref[...] / (den_ref[...] + 1e-16)
                  + LAMDA * dis_ref[...] * agg_ref[...]
                  + (1.0 - LAMDA) * h_ref[...] + b_ref[...])


def _assemble_emb(numer, den_col, agg, h, dis_col, b_row):
    grid = (N_NODES // 400,)
    return pl.pallas_call(
        _emb_body,
        grid=grid,
        in_specs=[
            pl.BlockSpec((400, D), lambda i: (i, 0)),
            pl.BlockSpec((400, 1), lambda i: (i, 0)),
            pl.BlockSpec((400, D), lambda i: (i, 0)),
            pl.BlockSpec((400, D), lambda i: (i, 0)),
            pl.BlockSpec((400, 1), lambda i: (i, 0)),
            pl.BlockSpec((1, D), lambda i: (0, 0)),
        ],
        out_specs=pl.BlockSpec((400, D), lambda i: (i, 0)),
        out_shape=jax.ShapeDtypeStruct((N_NODES, D), jnp.float32),
    )(numer, den_col, agg, h, dis_col, b_row)


def _norm_body(x_ref, o_ref):
    r = x_ref[...]
    nrm = jnp.sqrt(jnp.sum(r * r, axis=-1, keepdims=True))
    o_ref[...] = r / jnp.maximum(nrm, 1e-12)


def _normalize(rows):
    grid = (B // 512,)
    return pl.pallas_call(
        _norm_body,
        grid=grid,
        in_specs=[pl.BlockSpec((512, D), lambda i: (i, 0))],
        out_specs=pl.BlockSpec((512, D), lambda i: (i, 0)),
        out_shape=jax.ShapeDtypeStruct((B, D), jnp.float32),
    )(rows)


def kernel(x, edge_indices, edge_values, embedding, Wq, Wk, Wv, W, b):
    src = edge_indices[0].astype(jnp.int32)
    dst = edge_indices[1].astype(jnp.int32)
    pad = EPAD - E
    srcp = jnp.pad(src, (0, pad))
    dstp = jnp.pad(dst, (0, pad))
    evp = jnp.pad(edge_values.astype(jnp.float32), (0, pad))
    zeros_n = jnp.zeros((N16,), jnp.float32)
    zeros_nd = jnp.zeros((N16, D), jnp.float32)

    w4 = jnp.concatenate([Wq, Wk, Wv, W], axis=1).astype(jnp.float32)
    qkvh = _matmul4(embedding.astype(jnp.float32), w4)
    q = qkvh[:, 0:D]
    k = qkvh[:, D:2 * D]
    v = qkvh[:, 2 * D:3 * D]
    h = qkvh[:, 3 * D:4 * D]

    ew, deg_flat = _p1(q, k, srcp, dstp, evp, zeros_n)
    deg2 = deg_flat.reshape(NC, N16)
    dis_row = _compute_dis(deg2)
    dis_flat = dis_row[0, :N_NODES]
    dis_col = dis_flat[:, None]

    numer, den, agg = _p3(v, h, ew, srcp, dstp, evp, dis_flat,
                          zeros_nd, zeros_n)
    den_col = den[:N_NODES, None]

    emb = _assemble_emb(numer[:N_NODES], den_col, agg[:N_NODES], h, dis_col,
                        b.astype(jnp.float32)[None, :])
    outr = _p5(emb, x.astype(jnp.int32))
    return _normalize(outr)


# speedup vs baseline: 9.9610x; 1.2360x over previous
import functools

import jax
import jax.numpy as jnp
from jax import lax
from jax.experimental import pallas as pl
from jax.experimental.pallas import tpu as pltpu
from jax.experimental.pallas import tpu_sc as plsc

N_NODES = 10000
E = 320000
D = 128
B = 16384
LAMDA = 0.8

NC = 2
NS = 16
NW = NC * NS
CH = 128

NCH1 = 80
EPW = NCH1 * CH
EPAD = NW * EPW
NCHT = EPAD // CH
NCH3 = NCHT // NS
N16 = 10240

_INV_SQRT_D = 1.0 / (D ** 0.5)

_mesh = plsc.VectorSubcoreMesh(core_axis_name="c", subcore_axis_name="s")


def _copy_i32(src2d, row, dst1d):
    for g in range(CH // 16):
        sl = pl.ds(g * 16, 16)
        dst1d[sl] = src2d[row, sl]


@functools.partial(
    pl.kernel,
    out_type=[
        jax.ShapeDtypeStruct((EPAD,), jnp.float32),
        jax.ShapeDtypeStruct((NC * N16,), jnp.float32),
    ],
    mesh=_mesh,
    compiler_params=pltpu.CompilerParams(needs_layout_passes=False),
    scratch_types=[
        pltpu.VMEM((3, CH), jnp.int32),
        pltpu.VMEM((3, CH), jnp.int32),
        pltpu.VMEM((CH, D), jnp.float32),
        pltpu.VMEM((CH, D), jnp.float32),
        pltpu.VMEM((CH, D), jnp.float32),
        pltpu.VMEM((CH, D), jnp.float32),
        pltpu.VMEM((CH,), jnp.float32),
        pltpu.VMEM((CH,), jnp.float32),
        pltpu.VMEM((CH,), jnp.float32),
        pltpu.VMEM((CH,), jnp.float32),
        pltpu.VMEM((CH,), jnp.int32),
        pltpu.VMEM((CH,), jnp.int32),
        pltpu.VMEM_SHARED((N16,), jnp.float32),
    ] + [pltpu.SemaphoreType.DMA] * 10,
)
def _p1(q_hbm, k_hbm, edata_hbm, zn_hbm,
        ew_hbm, deg_hbm,
        ed0, ed1, qr0, qr1, kr0, kr1, ewv0, ewv1, evf0, evf1, dst0, dst1,
        deg_sh,
        sed0, sed1, sq0, sq1, sk0, sk1, sew0, sew1, sdg0, sdg1):
    c = lax.axis_index("c")
    s = lax.axis_index("s")
    wid = s * NC + c
    cbase = wid * NCH1

    ED = [ed0, ed1]
    QR = [qr0, qr1]
    KR = [kr0, kr1]
    EWV = [ewv0, ewv1]
    EVF = [evf0, evf1]
    DST = [dst0, dst1]
    SED = [sed0, sed1]
    SQ = [sq0, sq1]
    SK = [sk0, sk1]
    SEW = [sew0, sew1]
    SDG = [sdg0, sdg1]

    pltpu.sync_copy(zn_hbm.at[pl.ds(s * 640, 640)],
                    deg_sh.at[pl.ds(s * 640, 640)])
    plsc.subcore_barrier()

    iota = lax.iota(jnp.int32, 16)

    def issue_gathers(b):
        pltpu.async_copy(q_hbm.at[ED[b].at[0]], QR[b], SQ[b])
        pltpu.async_copy(k_hbm.at[ED[b].at[1]], KR[b], SK[b])

    pltpu.sync_copy(edata_hbm.at[cbase], ED[0])
    issue_gathers(0)
    pltpu.async_copy(edata_hbm.at[cbase + 1], ED[1], SED[1])

    def pair_body(i, _):
        for b in (0, 1):
            ci = 2 * i + b
            o = b ^ 1
            pltpu.make_async_copy(q_hbm.at[ED[b].at[0]], QR[b], SQ[b]).wait()
            pltpu.make_async_copy(k_hbm.at[ED[b].at[1]], KR[b], SK[b]).wait()

            @pl.when(ci >= 2)
            def _():
                pltpu.make_async_copy(
                    EVF[b], deg_sh.at[DST[b]], SDG[b]).wait()

            _copy_i32(ED[b], 0, DST[b])
            for g in range(CH // 16):
                sl = pl.ds(g * 16, 16)
                EVF[b][sl] = plsc.bitcast(ED[b][2, sl], jnp.float32)
            pltpu.async_copy(EVF[b], deg_sh.at[DST[b]], SDG[b], add=True)

            @pl.when(ci + 2 < NCH1)
            def _():
                pltpu.async_copy(edata_hbm.at[cbase + ci + 2], ED[b], SED[b])

            @pl.when(ci + 1 < NCH1)
            def _():
                pltpu.make_async_copy(
                    edata_hbm.at[cbase + ci + 1], ED[o], SED[o]).wait()
                issue_gathers(o)

            @pl.when(ci >= 2)
            def _():
                pltpu.make_async_copy(
                    EWV[b], ew_hbm.at[pl.ds(0, CH)], SEW[b]).wait()

            base = (cbase + ci) * CH

            def grp_body(g, _):
                lg = jnp.zeros((16,), jnp.float32)
                for i2 in range(16):
                    e = g * 16 + i2
                    acc = QR[b][e, pl.ds(0, 16)] * KR[b][e, pl.ds(0, 16)]
                    for j in range(1, D // 16):
                        sl = pl.ds(j * 16, 16)
                        acc = acc + QR[b][e, sl] * KR[b][e, sl]
                    lg = jnp.where(iota == i2, jnp.sum(acc), lg)
                ew = jnp.exp(lg * _INV_SQRT_D)
                eid = base + g * 16 + iota
                ew = jnp.where(eid < E, ew, 0.0)
                EWV[b][pl.ds(g * 16, 16)] = ew
                return 0

            lax.fori_loop(0, CH // 16, grp_body, 0)
            pltpu.async_copy(EWV[b], ew_hbm.at[pl.ds(base, CH)], SEW[b])
        return 0

    lax.fori_loop(0, NCH1 // 2, pair_body, 0)

    for b in (0, 1):
        pltpu.make_async_copy(EVF[b], deg_sh.at[DST[b]], SDG[b]).wait()
        pltpu.make_async_copy(EWV[b], ew_hbm.at[pl.ds(0, CH)], SEW[b]).wait()

    plsc.subcore_barrier()
    pltpu.sync_copy(deg_sh.at[pl.ds(s * 640, 640)],
                    deg_hbm.at[pl.ds(c * N16 + s * 640, 640)])


@functools.partial(
    pl.kernel,
    out_type=[
        jax.ShapeDtypeStruct((N16, D), jnp.float32),
        jax.ShapeDtypeStruct((N16,), jnp.float32),
        jax.ShapeDtypeStruct((N16, D), jnp.float32),
    ],
    mesh=_mesh,
    compiler_params=pltpu.CompilerParams(needs_layout_passes=False),
    scratch_types=[
        pltpu.VMEM((3, CH), jnp.int32),
        pltpu.VMEM((3, CH), jnp.int32),
        pltpu.VMEM((CH, D), jnp.float32),
        pltpu.VMEM((CH, D), jnp.float32),
        pltpu.VMEM((CH,), jnp.float32),
        pltpu.VMEM((CH,), jnp.float32),
        pltpu.VMEM((CH,), jnp.int32),
        pltpu.VMEM((CH,), jnp.int32),
        pltpu.VMEM_SHARED((N16, D), jnp.float32),
        pltpu.VMEM_SHARED((N16,), jnp.float32),
    ] + [pltpu.SemaphoreType.DMA] * 10,
)
def _p3(v_hbm, h_hbm, ew_hbm, edata_hbm, dis_hbm, znd_hbm, zn_hbm,
        numer_hbm, den_hbm, agg_hbm,
        ed0, ed1, rw0, rw1, wv0, wv1, dst0, dst1,
        rows_sh, den_sh,
        sed0, sed1, sr0, sr1, sw0, sw1, sn0, sn1, sd0, sd1):
    c = lax.axis_index("c")
    s = lax.axis_index("s")
    cbase = s * NCH3

    ED = [ed0, ed1]
    RW = [rw0, rw1]
    WV = [wv0, wv1]
    DST = [dst0, dst1]
    SED = [sed0, sed1]
    SR = [sr0, sr1]
    SW = [sw0, sw1]
    SN = [sn0, sn1]
    SD = [sd0, sd1]

    pltpu.sync_copy(znd_hbm.at[pl.ds(s * 640, 640)],
                    rows_sh.at[pl.ds(s * 640, 640)])
    pltpu.sync_copy(zn_hbm.at[pl.ds(s * 640, 640)],
                    den_sh.at[pl.ds(s * 640, 640)])
    plsc.subcore_barrier()

    def scale_rows(b):
        def g_body(g, _):
            w16 = WV[b][pl.ds(g * 16, 16)]
            for i2 in range(16):
                e = g * 16 + i2
                w = w16[i2]
                for j in range(D // 16):
                    sl = pl.ds(j * 16, 16)
                    RW[b][e, sl] = RW[b][e, sl] * w
            return 0
        lax.fori_loop(0, CH // 16, g_body, 0)

    def make_loop(is_attn):
        rows_tab = v_hbm if is_attn else h_hbm

        def issue_gathers(b, ci):
            pltpu.async_copy(rows_tab.at[ED[b].at[1]], RW[b], SR[b])
            if is_attn:
                pltpu.async_copy(ew_hbm.at[pl.ds((cbase + ci) * CH, CH)],
                                 WV[b], SW[b])
            else:
                pltpu.async_copy(dis_hbm.at[ED[b].at[1]], WV[b], SW[b])

        def wait_scatters(b):
            pltpu.make_async_copy(RW[b], rows_sh.at[DST[b]], SN[b]).wait()
            if is_attn:
                pltpu.make_async_copy(WV[b], den_sh.at[DST[b]], SD[b]).wait()

        pltpu.sync_copy(edata_hbm.at[cbase], ED[0])
        issue_gathers(0, 0)
        pltpu.async_copy(edata_hbm.at[cbase + 1], ED[1], SED[1])

        def pair_body(i, _):
            for b in (0, 1):
                ci = 2 * i + b
                o = b ^ 1
                pltpu.make_async_copy(rows_tab.at[ED[b].at[1]],
                                      RW[b], SR[b]).wait()
                if is_attn:
                    pltpu.make_async_copy(
                        ew_hbm.at[pl.ds(0, CH)], WV[b], SW[b]).wait()
                else:
                    pltpu.make_async_copy(dis_hbm.at[ED[b].at[1]],
                                          WV[b], SW[b]).wait()
                    for g in range(CH // 16):
                        sl = pl.ds(g * 16, 16)
                        WV[b][sl] = WV[b][sl] * plsc.bitcast(
                            ED[b][2, sl], jnp.float32)
                scale_rows(b)
                _copy_i32(ED[b], 0, DST[b])
                pltpu.async_copy(RW[b], rows_sh.at[DST[b]], SN[b], add=True)
                if is_attn:
                    pltpu.async_copy(WV[b], den_sh.at[DST[b]], SD[b],
                                     add=True)

                @pl.when(ci + 2 < NCH3)
                def _():
                    pltpu.async_copy(edata_hbm.at[cbase + ci + 2],
                                     ED[b], SED[b])

                @pl.when(ci + 1 < NCH3)
                def _():
                    pltpu.make_async_copy(
                        edata_hbm.at[cbase + ci + 1], ED[o], SED[o]).wait()

                    @pl.when(ci >= 1)
                    def _():
                        wait_scatters(o)
                    issue_gathers(o, ci + 1)
            return 0

        lax.fori_loop(0, NCH3 // 2, pair_body, 0)
        for b in (0, 1):
            wait_scatters(b)

    @pl.when(c == 0)
    def _():
        make_loop(True)

    @pl.when(c == 1)
    def _():
        make_loop(False)

    plsc.subcore_barrier()

    @pl.when(c == 0)
    def _():
        pltpu.sync_copy(rows_sh.at[pl.ds(s * 640, 640)],
                        numer_hbm.at[pl.ds(s * 640, 640)])
        pltpu.sync_copy(den_sh.at[pl.ds(s * 640, 640)],
                        den_hbm.at[pl.ds(s * 640, 640)])

    @pl.when(c == 1)
    def _():
        pltpu.sync_copy(rows_sh.at[pl.ds(s * 640, 640)],
                        agg_hbm.at[pl.ds(s * 640, 640)])


_BPW = B // NW
_BCH = _BPW // CH


@functools.partial(
    pl.kernel,
    out_type=jax.ShapeDtypeStruct((B, D), jnp.float32),
    mesh=_mesh,
    compiler_params=pltpu.CompilerParams(needs_layout_passes=False),
    scratch_types=[
        pltpu.VMEM((CH,), jnp.int32),
        pltpu.VMEM((CH, D), jnp.float32),
        pltpu.SemaphoreType.DMA,
    ],
)
def _p5(emb_hbm, x_hbm, out_hbm, xi, rows, sem):
    c = lax.axis_index("c")
    s = lax.axis_index("s")
    wid = s * NC + c

    def chunk_body(ci, _):
        base = wid * _BPW + ci * CH
        pltpu.sync_copy(x_hbm.at[pl.ds(base, CH)], xi)
        pltpu.async_copy(emb_hbm.at[xi], rows, sem).wait()
        pltpu.sync_copy(rows, out_hbm.at[pl.ds(base, CH)])
        return 0

    lax.fori_loop(0, _BCH, chunk_body, 0)


def _mm_body(a_ref, w_ref, o_ref):
    o_ref[...] = jnp.dot(a_ref[...], w_ref[...],
                         preferred_element_type=jnp.float32)


def _matmul4(embedding, w4):
    grid = (N_NODES // 400,)
    return pl.pallas_call(
        _mm_body,
        grid=grid,
        in_specs=[
            pl.BlockSpec((400, D), lambda i: (i, 0)),
            pl.BlockSpec((D, 4 * D), lambda i: (0, 0)),
        ],
        out_specs=pl.BlockSpec((400, 4 * D), lambda i: (i, 0)),
        out_shape=jax.ShapeDtypeStruct((N_NODES, 4 * D), jnp.float32),
    )(embedding, w4)


def _dis_body(deg_ref, dis_ref):
    d = deg_ref[0, :] + deg_ref[1, :]
    dis_ref[0, :] = jnp.where(d > 0, lax.rsqrt(jnp.where(d > 0, d, 1.0)), 0.0)


def _compute_dis(deg2):
    return pl.pallas_call(
        _dis_body,
        out_shape=jax.ShapeDtypeStruct((1, N16), jnp.float32),
    )(deg2)


def _emb_body(num_ref, den_ref, agg_ref, h_ref, dis_ref, b_ref, o_ref):
    o_ref[...] = (num_ref[...] / (den_ref[...] + 1e-16)
                  + LAMDA * dis_ref[...] * agg_ref[...]
                  + (1.0 - LAMDA) * h_ref[...] + b_ref[...])


def _assemble_emb(numer, den_col, agg, h, dis_col, b_row):
    grid = (N_NODES // 400,)
    return pl.pallas_call(
        _emb_body,
        grid=grid,
        in_specs=[
            pl.BlockSpec((400, D), lambda i: (i, 0)),
            pl.BlockSpec((400, 1), lambda i: (i, 0)),
            pl.BlockSpec((400, D), lambda i: (i, 0)),
            pl.BlockSpec((400, D), lambda i: (i, 0)),
            pl.BlockSpec((400, 1), lambda i: (i, 0)),
            pl.BlockSpec((1, D), lambda i: (0, 0)),
        ],
        out_specs=pl.BlockSpec((400, D), lambda i: (i, 0)),
        out_shape=jax.ShapeDtypeStruct((N_NODES, D), jnp.float32),
    )(numer, den_col, agg, h, dis_col, b_row)


def _norm_body(x_ref, o_ref):
    r = x_ref[...]
    nrm = jnp.sqrt(jnp.sum(r * r, axis=-1, keepdims=True))
    o_ref[...] = r / jnp.maximum(nrm, 1e-12)


def _normalize(rows):
    grid = (B // 512,)
    return pl.pallas_call(
        _norm_body,
        grid=grid,
        in_specs=[pl.BlockSpec((512, D), lambda i: (i, 0))],
        out_specs=pl.BlockSpec((512, D), lambda i: (i, 0)),
        out_shape=jax.ShapeDtypeStruct((B, D), jnp.float32),
    )(rows)


def kernel(x, edge_indices, edge_values, embedding, Wq, Wk, Wv, W, b):
    src = edge_indices[0].astype(jnp.int32)
    dst = edge_indices[1].astype(jnp.int32)
    pad = EPAD - E
    srcp = jnp.pad(src, (0, pad))
    dstp = jnp.pad(dst, (0, pad))
    evp = jnp.pad(edge_values.astype(jnp.float32), (0, pad))
    edata = jnp.stack([dstp, srcp,
                       lax.bitcast_convert_type(evp, jnp.int32)])
    edata = edata.reshape(3, NCHT, CH).transpose(1, 0, 2)
    zeros_n = jnp.zeros((N16,), jnp.float32)
    zeros_nd = jnp.zeros((N16, D), jnp.float32)

    w4 = jnp.concatenate([Wq, Wk, Wv, W], axis=1).astype(jnp.float32)
    qkvh = _matmul4(embedding.astype(jnp.float32), w4)
    q = qkvh[:, 0:D]
    k = qkvh[:, D:2 * D]
    v = qkvh[:, 2 * D:3 * D]
    h = qkvh[:, 3 * D:4 * D]

    ew, deg_flat = _p1(q, k, edata, zeros_n)
    deg2 = deg_flat.reshape(NC, N16)
    dis_row = _compute_dis(deg2)
    dis_flat = dis_row[0, :N_NODES]
    dis_col = dis_flat[:, None]

    numer, den, agg = _p3(v, h, ew, edata, dis_flat, zeros_nd, zeros_n)
    den_col = den[:N_NODES, None]

    emb = _assemble_emb(numer[:N_NODES], den_col, agg[:N_NODES], h, dis_col,
                        b.astype(jnp.float32)[None, :])
    outr = _p5(emb, x.astype(jnp.int32))
    return _normalize(outr)
